# baseline (speedup 1.0000x reference)
import functools

import jax
import jax.numpy as jnp
from jax import lax
from jax.experimental import pallas as pl
from jax.experimental.pallas import tpu as pltpu
from jax.experimental.pallas import tpu_sc as plsc

_NC, _NS = 2, 16
_NW = _NC * _NS
_ROWS, _COLS, _OUTC = 16384, 2048, 128
_STRIDE = _COLS // _OUTC
_R = 16
_ROWS_W = _ROWS // _NW
_CHUNKS = _ROWS_W // _R
_IN_W = _R * _COLS
_OUT_W = _R * _OUTC

_mesh = plsc.VectorSubcoreMesh(core_axis_name="c", subcore_axis_name="s")


@functools.partial(
    pl.kernel,
    out_type=jax.ShapeDtypeStruct((_ROWS * _OUTC,), jnp.float32),
    mesh=_mesh,
    scratch_types=[
        pltpu.VMEM((_IN_W,), jnp.float32),
        pltpu.VMEM((_OUT_W,), jnp.float32),
    ],
    compiler_params=pltpu.CompilerParams(needs_layout_passes=False),
)
def _select(x_hbm, out_hbm, xin, yout):
    wid = lax.axis_index("s") * _NC + lax.axis_index("c")
    base_in = wid * (_ROWS_W * _COLS)
    base_out = wid * (_ROWS_W * _OUTC)
    lane = lax.iota(jnp.int32, 16) * _STRIDE

    def chunk(g, carry):
        pltpu.sync_copy(x_hbm.at[pl.ds(base_in + g * _IN_W, _IN_W)], xin)
        for v in range(_OUT_W // 16):
            idx = lane + (v * 16 * _STRIDE)
            yout[pl.ds(v * 16, 16)] = plsc.load_gather(xin, [idx])
        pltpu.sync_copy(yout, out_hbm.at[pl.ds(base_out + g * _OUT_W, _OUT_W)])
        return carry

    lax.fori_loop(0, _CHUNKS, chunk, 0)


def kernel(x):
    return _select(x.reshape(-1)).reshape(_ROWS, _OUTC)


# speedup vs baseline: 1.1768x; 1.1768x over previous
import functools

import jax
import jax.numpy as jnp
from jax import lax
from jax.experimental import pallas as pl
from jax.experimental.pallas import tpu as pltpu
from jax.experimental.pallas import tpu_sc as plsc

_NC, _NS = 2, 16
_NW = _NC * _NS
_ROWS, _COLS, _OUTC = 16384, 2048, 128
_STRIDE = _COLS // _OUTC
_R = 16
_ROWS_W = _ROWS // _NW
_CHUNKS = _ROWS_W // _R
_IN_W = _R * _COLS
_OUT_W = _R * _OUTC

_mesh = plsc.VectorSubcoreMesh(core_axis_name="c", subcore_axis_name="s")


@functools.partial(
    pl.kernel,
    out_type=jax.ShapeDtypeStruct((_ROWS * _OUTC,), jnp.float32),
    mesh=_mesh,
    scratch_types=[
        pltpu.VMEM((_IN_W,), jnp.float32),
        pltpu.VMEM((_IN_W,), jnp.float32),
        pltpu.VMEM((_OUT_W,), jnp.float32),
        pltpu.VMEM((_OUT_W,), jnp.float32),
        pltpu.SemaphoreType.DMA,
        pltpu.SemaphoreType.DMA,
        pltpu.SemaphoreType.DMA,
        pltpu.SemaphoreType.DMA,
    ],
    compiler_params=pltpu.CompilerParams(needs_layout_passes=False),
)
def _select(x_hbm, out_hbm, xin0, xin1, yout0, yout1, si0, si1, so0, so1):
    wid = lax.axis_index("s") * _NC + lax.axis_index("c")
    base_in = wid * (_ROWS_W * _COLS)
    base_out = wid * (_ROWS_W * _OUTC)
    lane = lax.iota(jnp.int32, 16) * _STRIDE
    xins, youts = [xin0, xin1], [yout0, yout1]
    sis, sos = [si0, si1], [so0, so1]

    def in_slice(g):
        return x_hbm.at[pl.ds(base_in + g * _IN_W, _IN_W)]

    def out_slice(g):
        return out_hbm.at[pl.ds(base_out + g * _OUT_W, _OUT_W)]

    pltpu.async_copy(in_slice(0), xin0, si0)

    def body2(h, carry):
        for b in range(2):
            g = h * 2 + b
            pltpu.make_async_copy(in_slice(g), xins[b], sis[b]).wait()

            @pl.when(g + 1 < _CHUNKS)
            def _():
                pltpu.async_copy(in_slice(g + 1), xins[1 - b], sis[1 - b])

            @pl.when(g >= 2)
            def _():
                pltpu.make_async_copy(youts[b], out_slice(g - 2), sos[b]).wait()

            for v in range(_OUT_W // 16):
                idx = lane + (v * 16 * _STRIDE)
                youts[b][pl.ds(v * 16, 16)] = plsc.load_gather(xins[b], [idx])

            pltpu.async_copy(youts[b], out_slice(g), sos[b])
        return carry

    lax.fori_loop(0, _CHUNKS // 2, body2, 0)

    pltpu.make_async_copy(yout0, out_slice(_CHUNKS - 2), so0).wait()
    pltpu.make_async_copy(yout1, out_slice(_CHUNKS - 1), so1).wait()


def kernel(x):
    return _select(x.reshape(-1)).reshape(_ROWS, _OUTC)


# speedup vs baseline: 1.3037x; 1.1078x over previous
import functools

import jax
import jax.numpy as jnp
from jax import lax
from jax.experimental import pallas as pl
from jax.experimental.pallas import tpu as pltpu
from jax.experimental.pallas import tpu_sc as plsc

_NC, _NS = 2, 16
_NW = _NC * _NS
_ROWS, _COLS, _OUTC = 16384, 2048, 128
_STRIDE = _COLS // _OUTC
_R = 8
_ROWS_W = _ROWS // _NW
_CHUNKS = _ROWS_W // _R
_IN_W = _R * _COLS
_OUT_W = _R * _OUTC
_NBUF = 4

_mesh = plsc.VectorSubcoreMesh(core_axis_name="c", subcore_axis_name="s")


@functools.partial(
    pl.kernel,
    out_type=jax.ShapeDtypeStruct((_ROWS * _OUTC,), jnp.float32),
    mesh=_mesh,
    scratch_types=[
        [pltpu.VMEM((_IN_W,), jnp.float32) for _ in range(_NBUF)],
        [pltpu.VMEM((_OUT_W,), jnp.float32) for _ in range(_NBUF)],
        [pltpu.SemaphoreType.DMA for _ in range(_NBUF)],
        [pltpu.SemaphoreType.DMA for _ in range(_NBUF)],
    ],
    compiler_params=pltpu.CompilerParams(needs_layout_passes=False),
)
def _select(x_hbm, out_hbm, xins, youts, sis, sos):
    wid = lax.axis_index("s") * _NC + lax.axis_index("c")
    base_in = wid * (_ROWS_W * _COLS)
    base_out = wid * (_ROWS_W * _OUTC)
    lane = lax.iota(jnp.int32, 16) * _STRIDE

    def in_slice(g):
        return x_hbm.at[pl.ds(base_in + g * _IN_W, _IN_W)]

    def out_slice(g):
        return out_hbm.at[pl.ds(base_out + g * _OUT_W, _OUT_W)]

    for b in range(_NBUF - 1):
        pltpu.async_copy(in_slice(b), xins[b], sis[b])

    def body(h, carry):
        for b in range(_NBUF):
            g = h * _NBUF + b
            pltpu.make_async_copy(in_slice(g), xins[b], sis[b]).wait()

            @pl.when(g + _NBUF - 1 < _CHUNKS)
            def _():
                nb = (b + _NBUF - 1) % _NBUF
                pltpu.async_copy(in_slice(g + _NBUF - 1), xins[nb], sis[nb])

            @pl.when(g >= _NBUF)
            def _():
                pltpu.make_async_copy(youts[b], out_slice(g - _NBUF), sos[b]).wait()

            for v in range(_OUT_W // 16):
                idx = lane + (v * 16 * _STRIDE)
                youts[b][pl.ds(v * 16, 16)] = plsc.load_gather(xins[b], [idx])

            pltpu.async_copy(youts[b], out_slice(g), sos[b])
        return carry

    lax.fori_loop(0, _CHUNKS // _NBUF, body, 0)

    for b in range(_NBUF):
        g = _CHUNKS - _NBUF + b
        pltpu.make_async_copy(youts[b], out_slice(g), sos[b]).wait()


def kernel(x):
    return _select(x.reshape(-1)).reshape(_ROWS, _OUTC)


# speedup vs baseline: 3.0530x; 2.3418x over previous
import functools

import jax
import jax.numpy as jnp
from jax import lax
from jax.experimental import pallas as pl
from jax.experimental.pallas import tpu as pltpu
from jax.experimental.pallas import tpu_sc as plsc

_NC, _NS = 2, 16
_NW = _NC * _NS
_ROWS, _COLS, _OUTC = 16384, 2048, 128
_STRIDE = _COLS // _OUTC
_R = 8
_ROWS_W = _ROWS // _NW
_CHUNKS = _ROWS_W // _R
_NBUF = 4

_mesh = plsc.VectorSubcoreMesh(core_axis_name="c", subcore_axis_name="s")


@functools.partial(
    pl.kernel,
    out_type=jax.ShapeDtypeStruct((_ROWS, _OUTC), jnp.float32),
    mesh=_mesh,
    scratch_types=[
        [pltpu.VMEM((_R, _COLS), jnp.float32) for _ in range(_NBUF)],
        [pltpu.VMEM((_R, _OUTC), jnp.float32) for _ in range(_NBUF)],
        [pltpu.SemaphoreType.DMA for _ in range(_NBUF)],
        [pltpu.SemaphoreType.DMA for _ in range(_NBUF)],
    ],
    compiler_params=pltpu.CompilerParams(needs_layout_passes=False),
)
def _select(x_hbm, out_hbm, xins, youts, sis, sos):
    wid = lax.axis_index("s") * _NC + lax.axis_index("c")
    row0 = wid * _ROWS_W
    lane = lax.iota(jnp.int32, 16)
    col_sel = lane * _STRIDE

    def in_slice(g):
        return x_hbm.at[pl.ds(row0 + g * _R, _R), :]

    def out_slice(g):
        return out_hbm.at[pl.ds(row0 + g * _R, _R), :]

    for b in range(_NBUF - 1):
        pltpu.async_copy(in_slice(b), xins[b], sis[b])

    def body(h, carry):
        for b in range(_NBUF):
            g = h * _NBUF + b
            pltpu.make_async_copy(in_slice(g), xins[b], sis[b]).wait()

            @pl.when(g + _NBUF - 1 < _CHUNKS)
            def _():
                nb = (b + _NBUF - 1) % _NBUF
                pltpu.async_copy(in_slice(g + _NBUF - 1), xins[nb], sis[nb])

            @pl.when(g >= _NBUF)
            def _():
                pltpu.make_async_copy(youts[b], out_slice(g - _NBUF), sos[b]).wait()

            for r in range(_R):
                rvec = jnp.full((16,), r, jnp.int32)
                for v in range(_OUTC // 16):
                    idx_col = col_sel + (v * 16 * _STRIDE)
                    youts[b][r, pl.ds(v * 16, 16)] = plsc.load_gather(
                        xins[b], [rvec, idx_col]
                    )

            pltpu.async_copy(youts[b], out_slice(g), sos[b])
        return carry

    lax.fori_loop(0, _CHUNKS // _NBUF, body, 0)

    for b in range(_NBUF):
        g = _CHUNKS - _NBUF + b
        pltpu.make_async_copy(youts[b], out_slice(g), sos[b]).wait()


def kernel(x):
    return _select(x)
